# baseline (speedup 1.0000x reference)
import functools

import jax
import jax.numpy as jnp
from jax import lax
from jax.experimental import pallas as pl
from jax.experimental.pallas import tpu as pltpu
import jax.experimental.pallas.tpu_sc as plsc

_NC = 2
_NS = 16
_NW = _NC * _NS

_CH = 4


def _sc_gather(table, idx3):
  nw, nch, ch = idx3.shape
  d = table.shape[1]
  b_total = nw * nch * ch
  assert nch >= 2 and nch % 2 == 0
  mesh = plsc.VectorSubcoreMesh(core_axis_name="c", subcore_axis_name="s")

  @functools.partial(
      pl.kernel,
      out_type=jax.ShapeDtypeStruct((b_total, d), jnp.float32),
      mesh=mesh,
      scratch_types=[
          pltpu.VMEM((nch, ch), jnp.int32),
          pltpu.VMEM((ch, d), jnp.float32),
          pltpu.VMEM((ch, d), jnp.float32),
          pltpu.SemaphoreType.DMA,
          pltpu.SemaphoreType.DMA,
          pltpu.SemaphoreType.DMA,
          pltpu.SemaphoreType.DMA,
      ],
  )
  def k(table_hbm, idx_hbm, out_hbm, idx_v, buf0, buf1, gs0, gs1, ss0, ss1):
    bufs = (buf0, buf1)
    gsems = (gs0, gs1)
    ssems = (ss0, ss1)
    wid = lax.axis_index("s") * _NC + lax.axis_index("c")
    base_row = wid * (nch * ch)

    pltpu.sync_copy(idx_hbm.at[wid], idx_v)

    def gather_start(b, g):
      pltpu.async_copy(table_hbm.at[idx_v.at[g]], bufs[b], gsems[b])

    def gather_wait(b):
      pltpu.make_async_copy(table_hbm.at[idx_v.at[0]], bufs[b],
                            gsems[b]).wait()

    def scatter_start(b, g):
      pltpu.async_copy(bufs[b], out_hbm.at[pl.ds(base_row + g * ch, ch)],
                       ssems[b])

    def scatter_wait(b):
      pltpu.make_async_copy(bufs[b], out_hbm.at[pl.ds(0, ch)],
                            ssems[b]).wait()

    def step(g, b, first=False, last=False):
      gather_wait(b)
      scatter_start(b, g)
      if not first:
        scatter_wait(1 - b)
      if not last:
        gather_start(1 - b, g + 1)

    gather_start(0, 0)
    step(0, 0, first=True)

    @pl.loop(0, (nch - 2) // 2)
    def _(o):
      step(2 * o + 1, 1)
      step(2 * o + 2, 0)

    step(nch - 1, 1, last=True)
    scatter_wait(1)

  return k(table, idx3)


def kernel(x, table):
  b, t = x.shape
  vocab = table.shape[0]
  idx = x.reshape(-1).astype(jnp.int32)
  b_total = idx.shape[0]
  r = b_total // _NW
  idx3 = idx.reshape(_NW, r // _CH, _CH)
  out = _sc_gather(table, idx3)
  return out.reshape(b, t, vocab)
